# baseline (speedup 1.0000x reference)
import functools

import jax
import jax.numpy as jnp
from jax import lax
from jax.experimental import pallas as pl
from jax.experimental.pallas import tpu as pltpu
from jax.experimental.pallas import tpu_sc as plsc

_NC, _NS = 2, 16
_NW = _NC * _NS
_CHUNK = 80
_NBUF = 2
_NG = 64


def _dense1_body(x_ref, w_ref, bvec_ref, xw_ref, d_ref):
    y = jnp.dot(x_ref[...], w_ref[...], preferred_element_type=jnp.float32)
    yb = y + bvec_ref[...]
    xw_ref[...] = y[:, :64]
    a = jax.nn.relu(yb[:, 64:96])
    t = jnp.tanh(yb[:, 96:98]) * jnp.tanh(yb[:, 98:100])
    d_ref[...] = jnp.concatenate([a, t], axis=1)


def _dense1(x, w1p, bvec1, tr):
    n = x.shape[0]
    grid = n // tr
    return pl.pallas_call(
        _dense1_body,
        grid=(grid,),
        in_specs=[
            pl.BlockSpec((tr, 128), lambda i: (i, 0)),
            pl.BlockSpec((128, 128), lambda i: (0, 0)),
            pl.BlockSpec((1, 128), lambda i: (0, 0)),
        ],
        out_specs=[
            pl.BlockSpec((tr, 64), lambda i: (i, 0)),
            pl.BlockSpec((tr, 34), lambda i: (i, 0)),
        ],
        out_shape=[
            jax.ShapeDtypeStruct((n, 64), jnp.float32),
            jax.ShapeDtypeStruct((n, 34), jnp.float32),
        ],
        compiler_params=pltpu.CompilerParams(
            dimension_semantics=("parallel",)),
    )(x, w1p, bvec1)


def _dense2_body(d_ref, p_ref, bc_ref, bns_ref, bnb_ref, w_ref, bvec_ref,
                 xw_ref, d2_ref):
    spect = jax.nn.relu(p_ref[0] + p_ref[1] + bc_ref[...])
    tr = spect.shape[0]
    h1 = jnp.concatenate(
        [d_ref[:, :32], spect, d_ref[:, 32:34]], axis=1)
    h1 = h1 * bns_ref[...] + bnb_ref[...]
    h1p = jnp.concatenate([h1, jnp.zeros((tr, 30), jnp.float32)], axis=1)
    y = jnp.dot(h1p, w_ref[...], preferred_element_type=jnp.float32)
    yb = y + bvec_ref[...]
    xw_ref[...] = y[:, :64]
    a = jax.nn.relu(yb[:, 64:96])
    t = jnp.tanh(yb[:, 96:98]) * jnp.tanh(yb[:, 98:100])
    d2_ref[...] = jnp.concatenate([a, t], axis=1)


def _dense2(d1, p, bc11, bns, bnb, w2p, bvec2, tr):
    n = d1.shape[0]
    grid = n // tr
    return pl.pallas_call(
        _dense2_body,
        grid=(grid,),
        in_specs=[
            pl.BlockSpec((tr, 34), lambda i: (i, 0)),
            pl.BlockSpec((2, tr, 64), lambda i: (0, i, 0)),
            pl.BlockSpec((1, 64), lambda i: (0, 0)),
            pl.BlockSpec((1, 98), lambda i: (0, 0)),
            pl.BlockSpec((1, 98), lambda i: (0, 0)),
            pl.BlockSpec((128, 128), lambda i: (0, 0)),
            pl.BlockSpec((1, 128), lambda i: (0, 0)),
        ],
        out_specs=[
            pl.BlockSpec((tr, 64), lambda i: (i, 0)),
            pl.BlockSpec((tr, 34), lambda i: (i, 0)),
        ],
        out_shape=[
            jax.ShapeDtypeStruct((n, 64), jnp.float32),
            jax.ShapeDtypeStruct((n, 34), jnp.float32),
        ],
        compiler_params=pltpu.CompilerParams(
            dimension_semantics=("parallel",)),
    )(d1, p, bc11, bns, bnb, w2p, bvec2)


def _make_sc_agg(npad, steps, chunk):
    rows_per_tile = npad // _NS
    mesh = plsc.VectorSubcoreMesh(
        core_axis_name="c", subcore_axis_name="s",
        num_cores=_NC, num_subcores=_NS)

    @functools.partial(
        pl.kernel,
        out_type=jax.ShapeDtypeStruct((_NC, npad, 64), jnp.float32),
        mesh=mesh,
        scratch_types=[
            pltpu.VMEM((steps + _NBUF, chunk), jnp.int32),
            pltpu.VMEM((steps + _NBUF, chunk), jnp.int32),
            [pltpu.VMEM((chunk, 64), jnp.float32) for _ in range(_NBUF)],
            pltpu.VMEM_SHARED((npad, 64), jnp.float32),
            pltpu.VMEM_SHARED((npad, 64), jnp.float32),
            [pltpu.SemaphoreType.DMA for _ in range(_NBUF)],
        ],
        compiler_params=pltpu.CompilerParams(use_tc_tiling_on_sc=False),
    )
    def sc_agg(xw_hbm, src_hbm, dst_hbm, zeros_hbm, out_hbm,
               src_v, dst_v, rows, acc, tbl, gsem):
        c = lax.axis_index("c")
        s = lax.axis_index("s")
        wid = s * _NC + c
        r0 = s * rows_per_tile
        pltpu.sync_copy(zeros_hbm.at[pl.ds(r0, rows_per_tile)],
                        acc.at[pl.ds(r0, rows_per_tile)])
        pltpu.sync_copy(xw_hbm.at[pl.ds(r0, rows_per_tile)],
                        tbl.at[pl.ds(r0, rows_per_tile)])
        pltpu.sync_copy(src_hbm.at[wid], src_v)
        pltpu.sync_copy(dst_hbm.at[wid], dst_v)
        plsc.subcore_barrier()

        def rnd(i, carry):
            j0 = 2 * i
            j1 = 2 * i + 1
            cpb = pltpu.async_copy(xw_hbm.at[src_v.at[j1]], rows[1], gsem[1])
            pltpu.async_copy(tbl.at[src_v.at[j0]], rows[0], gsem[0]).wait()
            pltpu.sync_copy(rows[0], acc.at[dst_v.at[j0]], add=True)
            cpb.wait()
            pltpu.sync_copy(rows[1], acc.at[dst_v.at[j1]], add=True)
            return carry

        lax.fori_loop(0, steps // 2, rnd, 0)
        plsc.subcore_barrier()
        pltpu.sync_copy(acc.at[pl.ds(r0, rows_per_tile)],
                        out_hbm.at[c, pl.ds(r0, rows_per_tile)])

    return sc_agg


def _sc_agg_impl(xw, srcw, dstw, zeros, npad):
    steps = srcw.shape[1] - _NBUF
    return _make_sc_agg(npad, steps, srcw.shape[2])(xw, srcw, dstw, zeros)


def _pool_body(d_ref, q_ref, bc_ref, b_ref, w1_ref, b1_ref, w2_ref, b2_ref,
               o_ref, ps_ref, pm_ref):
    i = pl.program_id(0)

    @pl.when(i == 0)
    def _init():
        ps_ref[...] = jnp.zeros_like(ps_ref)
        pm_ref[...] = jnp.full_like(pm_ref, -jnp.inf)

    spect = jax.nn.relu(q_ref[0] + q_ref[1] + bc_ref[...])
    h2 = jnp.concatenate(
        [d_ref[:, :32], spect, d_ref[:, 32:34]], axis=1)
    b2 = b_ref[0]
    tr = h2.shape[0]
    onehot = (b2 == lax.broadcasted_iota(jnp.int32, (tr, _NG), 1)
              ).astype(jnp.float32)
    ps_ref[...] += lax.dot_general(
        onehot, h2, (((0,), (0,)), ((), ())),
        preferred_element_type=jnp.float32)
    m = h2
    dshift = 1
    while dshift < tr:
        mprev = jnp.concatenate(
            [jnp.full((dshift, 98), -jnp.inf, jnp.float32), m[:-dshift]],
            axis=0)
        bprev = jnp.concatenate(
            [jnp.full((dshift, 1), -1, jnp.int32), b2[:-dshift]], axis=0)
        m = jnp.where(bprev == b2, jnp.maximum(m, mprev), m)
        dshift *= 2
    bnext = jnp.concatenate(
        [b2[1:], jnp.full((1, 1), -1, jnp.int32)], axis=0)
    is_end = (b2 != bnext).astype(jnp.float32)
    sel = onehot * is_end
    contrib = lax.dot_general(
        sel, m, (((0,), (0,)), ((), ())),
        preferred_element_type=jnp.float32)
    present = jnp.sum(sel, axis=0)[:, None] > 0.0
    pm_ref[...] = jnp.maximum(
        pm_ref[...], jnp.where(present, contrib, -jnp.inf))

    @pl.when(i == pl.num_programs(0) - 1)
    def _head():
        pooled = jnp.concatenate([ps_ref[...], pm_ref[...]], axis=1)
        z1 = jax.nn.relu(
            jnp.dot(pooled, w1_ref[...], preferred_element_type=jnp.float32)
            + b1_ref[...])
        z = (jnp.dot(z1, w2_ref[...], preferred_element_type=jnp.float32)
             + b2_ref[...])
        m = jnp.max(z, axis=1, keepdims=True)
        lse = jnp.log(jnp.sum(jnp.exp(z - m), axis=1, keepdims=True)) + m
        o_ref[...] = z - lse


def _pool(d2, q, bc21, batch3, wfc1, bfc1, wfc2, bfc2, tr):
    n = d2.shape[0]
    grid = n // tr
    return pl.pallas_call(
        _pool_body,
        grid=(grid,),
        in_specs=[
            pl.BlockSpec((tr, 34), lambda i: (i, 0)),
            pl.BlockSpec((2, tr, 64), lambda i: (0, i, 0)),
            pl.BlockSpec((1, 64), lambda i: (0, 0)),
            pl.BlockSpec((1, tr, 1), lambda i: (i, 0, 0)),
            pl.BlockSpec((196, 100), lambda i: (0, 0)),
            pl.BlockSpec((1, 100), lambda i: (0, 0)),
            pl.BlockSpec((100, 2), lambda i: (0, 0)),
            pl.BlockSpec((1, 2), lambda i: (0, 0)),
        ],
        out_specs=pl.BlockSpec((_NG, 2), lambda i: (0, 0)),
        out_shape=jax.ShapeDtypeStruct((_NG, 2), jnp.float32),
        scratch_shapes=[
            pltpu.VMEM((_NG, 98), jnp.float32),
            pltpu.VMEM((_NG, 98), jnp.float32),
        ],
        compiler_params=pltpu.CompilerParams(
            dimension_semantics=("arbitrary",)),
    )(d2, q, bc21, batch3, wfc1, bfc1.reshape(1, -1), wfc2,
      bfc2.reshape(1, -1))


def kernel(x, edge_index, batch, Wc11, bc11, Wfc11, bfc11, Wfc12, bfc12,
           Wfc13, bfc13, bn1_g, bn1_b, Wc21, bc21, Wfc21, bfc21, Wfc22,
           bfc22, Wfc23, bfc23, Wfc1, bfc1, Wfc2, bfc2):
    n, f_in = x.shape
    e = edge_index.shape[1]
    tr = 400
    z32 = jnp.zeros((32,), jnp.float32)

    w1p = jnp.concatenate([Wc11, Wfc11, Wfc12, Wfc13,
                           jnp.zeros((f_in, 28), jnp.float32)], axis=1)
    bvec1 = jnp.concatenate([jnp.zeros((64,), jnp.float32), bfc11, bfc12,
                             bfc13, jnp.zeros((28,), jnp.float32)]
                            ).reshape(1, 128)
    w2p = jnp.concatenate([Wc21, Wfc21, Wfc22, Wfc23,
                           jnp.zeros((98, 28), jnp.float32)], axis=1)
    w2p = jnp.concatenate([w2p, jnp.zeros((30, 128), jnp.float32)], axis=0)
    bvec2 = jnp.concatenate([jnp.zeros((64,), jnp.float32), bfc21, bfc22,
                             bfc23, jnp.zeros((28,), jnp.float32)]
                            ).reshape(1, 128)
    bns = (bn1_g / jnp.sqrt(1.0 + 1e-5)).reshape(1, 98)
    bnb = bn1_b.reshape(1, 98)

    per = _NW * _CHUNK * _NBUF
    e_pad = ((e + per - 1) // per) * per
    npad = ((n + 127) // 128) * 128
    if npad == n:
        npad += 128
    src = edge_index[0]
    dst = edge_index[1]
    if e_pad != e:
        fill = jnp.full((e_pad - e,), n, jnp.int32)
        src = jnp.concatenate([src, fill])
        dst = jnp.concatenate([dst, fill])
    steps = e_pad // (_NW * _CHUNK)
    dummy = jnp.full((_NW, _NBUF, _CHUNK), n, jnp.int32)
    srcw = jnp.concatenate([src.reshape(_NW, steps, _CHUNK), dummy], axis=1)
    dstw = jnp.concatenate([dst.reshape(_NW, steps, _CHUNK), dummy], axis=1)
    zeros = jnp.zeros((npad, 64), jnp.float32)

    def pad_rows(a):
        if npad == n:
            return a
        return jnp.concatenate(
            [a, jnp.zeros((npad - n, 64), jnp.float32)], axis=0)

    xw1, d1 = _dense1(x, w1p, bvec1, tr)
    p = _sc_agg_impl(pad_rows(xw1), srcw, dstw, zeros, npad)
    xw2, d2 = _dense2(d1, p[:, :n], bc11.reshape(1, 64), bns, bnb,
                      w2p, bvec2, tr)
    q = _sc_agg_impl(pad_rows(xw2), srcw, dstw, zeros, npad)
    batch3 = batch.reshape(n // tr, tr, 1)
    return _pool(d2, q[:, :n], bc21.reshape(1, 64), batch3,
                 Wfc1, bfc1, Wfc2, bfc2, tr)


# speedup vs baseline: 1.1582x; 1.1582x over previous
import functools

import jax
import jax.numpy as jnp
from jax import lax
from jax.experimental import pallas as pl
from jax.experimental.pallas import tpu as pltpu
from jax.experimental.pallas import tpu_sc as plsc

_NC, _NS = 2, 16
_NW = _NC * _NS
_CHUNK = 80
_NBUF = 1
_NG = 64


def _dense1_body(x_ref, w_ref, bvec_ref, xw_ref, d_ref):
    y = jnp.dot(x_ref[...], w_ref[...], preferred_element_type=jnp.float32)
    yb = y + bvec_ref[...]
    xw_ref[...] = y[:, :64]
    a = jax.nn.relu(yb[:, 64:96])
    t = jnp.tanh(yb[:, 96:98]) * jnp.tanh(yb[:, 98:100])
    d_ref[...] = jnp.concatenate([a, t], axis=1)


def _dense1(x, w1p, bvec1, tr):
    n = x.shape[0]
    grid = n // tr
    return pl.pallas_call(
        _dense1_body,
        grid=(grid,),
        in_specs=[
            pl.BlockSpec((tr, 128), lambda i: (i, 0)),
            pl.BlockSpec((128, 128), lambda i: (0, 0)),
            pl.BlockSpec((1, 128), lambda i: (0, 0)),
        ],
        out_specs=[
            pl.BlockSpec((tr, 64), lambda i: (i, 0)),
            pl.BlockSpec((tr, 34), lambda i: (i, 0)),
        ],
        out_shape=[
            jax.ShapeDtypeStruct((n, 64), jnp.float32),
            jax.ShapeDtypeStruct((n, 34), jnp.float32),
        ],
        compiler_params=pltpu.CompilerParams(
            dimension_semantics=("parallel",)),
    )(x, w1p, bvec1)


def _dense2_body(d_ref, p_ref, bc_ref, bns_ref, bnb_ref, w_ref, bvec_ref,
                 xw_ref, d2_ref):
    spect = jax.nn.relu(p_ref[0] + p_ref[1] + bc_ref[...])
    tr = spect.shape[0]
    h1 = jnp.concatenate(
        [d_ref[:, :32], spect, d_ref[:, 32:34]], axis=1)
    h1 = h1 * bns_ref[...] + bnb_ref[...]
    h1p = jnp.concatenate([h1, jnp.zeros((tr, 30), jnp.float32)], axis=1)
    y = jnp.dot(h1p, w_ref[...], preferred_element_type=jnp.float32)
    yb = y + bvec_ref[...]
    xw_ref[...] = y[:, :64]
    a = jax.nn.relu(yb[:, 64:96])
    t = jnp.tanh(yb[:, 96:98]) * jnp.tanh(yb[:, 98:100])
    d2_ref[...] = jnp.concatenate([a, t], axis=1)


def _dense2(d1, p, bc11, bns, bnb, w2p, bvec2, tr):
    n = d1.shape[0]
    grid = n // tr
    return pl.pallas_call(
        _dense2_body,
        grid=(grid,),
        in_specs=[
            pl.BlockSpec((tr, 34), lambda i: (i, 0)),
            pl.BlockSpec((2, tr, 64), lambda i: (0, i, 0)),
            pl.BlockSpec((1, 64), lambda i: (0, 0)),
            pl.BlockSpec((1, 98), lambda i: (0, 0)),
            pl.BlockSpec((1, 98), lambda i: (0, 0)),
            pl.BlockSpec((128, 128), lambda i: (0, 0)),
            pl.BlockSpec((1, 128), lambda i: (0, 0)),
        ],
        out_specs=[
            pl.BlockSpec((tr, 64), lambda i: (i, 0)),
            pl.BlockSpec((tr, 34), lambda i: (i, 0)),
        ],
        out_shape=[
            jax.ShapeDtypeStruct((n, 64), jnp.float32),
            jax.ShapeDtypeStruct((n, 34), jnp.float32),
        ],
        compiler_params=pltpu.CompilerParams(
            dimension_semantics=("parallel",)),
    )(d1, p, bc11, bns, bnb, w2p, bvec2)


def _make_sc_agg(npad, steps, chunk):
    rows_per_tile = npad // _NS
    mesh = plsc.VectorSubcoreMesh(
        core_axis_name="c", subcore_axis_name="s",
        num_cores=_NC, num_subcores=_NS)

    @functools.partial(
        pl.kernel,
        out_type=jax.ShapeDtypeStruct((_NC, npad, 64), jnp.float32),
        mesh=mesh,
        scratch_types=[
            pltpu.VMEM((steps + _NBUF, chunk), jnp.int32),
            pltpu.VMEM((steps + _NBUF, chunk), jnp.int32),
            [pltpu.VMEM((chunk, 64), jnp.float32) for _ in range(_NBUF)],
            pltpu.VMEM_SHARED((npad, 64), jnp.float32),
            pltpu.VMEM_SHARED((npad, 64), jnp.float32),
            [pltpu.SemaphoreType.DMA for _ in range(_NBUF)],
        ],
        compiler_params=pltpu.CompilerParams(use_tc_tiling_on_sc=False),
    )
    def sc_agg(xw_hbm, src_hbm, dst_hbm, zeros_hbm, out_hbm,
               src_v, dst_v, rows, acc, tbl, gsem):
        c = lax.axis_index("c")
        s = lax.axis_index("s")
        wid = s * _NC + c
        r0 = s * rows_per_tile
        pltpu.sync_copy(zeros_hbm.at[pl.ds(r0, rows_per_tile)],
                        acc.at[pl.ds(r0, rows_per_tile)])
        pltpu.sync_copy(xw_hbm.at[pl.ds(r0, rows_per_tile)],
                        tbl.at[pl.ds(r0, rows_per_tile)])
        pltpu.sync_copy(src_hbm.at[wid], src_v)
        pltpu.sync_copy(dst_hbm.at[wid], dst_v)
        plsc.subcore_barrier()

        for b in range(_NBUF):
            pltpu.async_copy(tbl.at[src_v.at[b]], rows[b], gsem[b])

        def rnd(i, carry):
            j0 = i * _NBUF
            for b in range(_NBUF):
                pltpu.make_async_copy(
                    tbl.at[src_v.at[j0 + b]], rows[b], gsem[b]).wait()
                pltpu.sync_copy(rows[b], acc.at[dst_v.at[j0 + b]], add=True)
                pltpu.async_copy(tbl.at[src_v.at[j0 + _NBUF + b]],
                                 rows[b], gsem[b])
            return carry

        lax.fori_loop(0, steps // _NBUF, rnd, 0)
        for b in range(_NBUF):
            pltpu.make_async_copy(
                tbl.at[src_v.at[b]], rows[b], gsem[b]).wait()
        plsc.subcore_barrier()
        pltpu.sync_copy(acc.at[pl.ds(r0, rows_per_tile)],
                        out_hbm.at[c, pl.ds(r0, rows_per_tile)])

    return sc_agg


def _sc_agg_impl(xw, srcw, dstw, zeros, npad):
    steps = srcw.shape[1] - _NBUF
    return _make_sc_agg(npad, steps, srcw.shape[2])(xw, srcw, dstw, zeros)


def _pool_body(d_ref, q_ref, bc_ref, b_ref, w1_ref, b1_ref, w2_ref, b2_ref,
               o_ref, ps_ref, pm_ref):
    i = pl.program_id(0)

    @pl.when(i == 0)
    def _init():
        ps_ref[...] = jnp.zeros_like(ps_ref)
        pm_ref[...] = jnp.full_like(pm_ref, -jnp.inf)

    spect = jax.nn.relu(q_ref[0] + q_ref[1] + bc_ref[...])
    h2 = jnp.concatenate(
        [d_ref[:, :32], spect, d_ref[:, 32:34]], axis=1)
    b2 = b_ref[0]
    tr = h2.shape[0]
    onehot = (b2 == lax.broadcasted_iota(jnp.int32, (tr, _NG), 1)
              ).astype(jnp.float32)
    ps_ref[...] += lax.dot_general(
        onehot, h2, (((0,), (0,)), ((), ())),
        preferred_element_type=jnp.float32)
    m = h2
    dshift = 1
    while dshift < tr:
        mprev = jnp.concatenate(
            [jnp.full((dshift, 98), -jnp.inf, jnp.float32), m[:-dshift]],
            axis=0)
        bprev = jnp.concatenate(
            [jnp.full((dshift, 1), -1, jnp.int32), b2[:-dshift]], axis=0)
        m = jnp.where(bprev == b2, jnp.maximum(m, mprev), m)
        dshift *= 2
    bnext = jnp.concatenate(
        [b2[1:], jnp.full((1, 1), -1, jnp.int32)], axis=0)
    is_end = (b2 != bnext).astype(jnp.float32)
    sel = onehot * is_end
    contrib = lax.dot_general(
        sel, m, (((0,), (0,)), ((), ())),
        preferred_element_type=jnp.float32)
    present = jnp.sum(sel, axis=0)[:, None] > 0.0
    pm_ref[...] = jnp.maximum(
        pm_ref[...], jnp.where(present, contrib, -jnp.inf))

    @pl.when(i == pl.num_programs(0) - 1)
    def _head():
        pooled = jnp.concatenate([ps_ref[...], pm_ref[...]], axis=1)
        z1 = jax.nn.relu(
            jnp.dot(pooled, w1_ref[...], preferred_element_type=jnp.float32)
            + b1_ref[...])
        z = (jnp.dot(z1, w2_ref[...], preferred_element_type=jnp.float32)
             + b2_ref[...])
        m = jnp.max(z, axis=1, keepdims=True)
        lse = jnp.log(jnp.sum(jnp.exp(z - m), axis=1, keepdims=True)) + m
        o_ref[...] = z - lse


def _pool(d2, q, bc21, batch3, wfc1, bfc1, wfc2, bfc2, tr):
    n = d2.shape[0]
    grid = n // tr
    return pl.pallas_call(
        _pool_body,
        grid=(grid,),
        in_specs=[
            pl.BlockSpec((tr, 34), lambda i: (i, 0)),
            pl.BlockSpec((2, tr, 64), lambda i: (0, i, 0)),
            pl.BlockSpec((1, 64), lambda i: (0, 0)),
            pl.BlockSpec((1, tr, 1), lambda i: (i, 0, 0)),
            pl.BlockSpec((196, 100), lambda i: (0, 0)),
            pl.BlockSpec((1, 100), lambda i: (0, 0)),
            pl.BlockSpec((100, 2), lambda i: (0, 0)),
            pl.BlockSpec((1, 2), lambda i: (0, 0)),
        ],
        out_specs=pl.BlockSpec((_NG, 2), lambda i: (0, 0)),
        out_shape=jax.ShapeDtypeStruct((_NG, 2), jnp.float32),
        scratch_shapes=[
            pltpu.VMEM((_NG, 98), jnp.float32),
            pltpu.VMEM((_NG, 98), jnp.float32),
        ],
        compiler_params=pltpu.CompilerParams(
            dimension_semantics=("arbitrary",)),
    )(d2, q, bc21, batch3, wfc1, bfc1.reshape(1, -1), wfc2,
      bfc2.reshape(1, -1))


def kernel(x, edge_index, batch, Wc11, bc11, Wfc11, bfc11, Wfc12, bfc12,
           Wfc13, bfc13, bn1_g, bn1_b, Wc21, bc21, Wfc21, bfc21, Wfc22,
           bfc22, Wfc23, bfc23, Wfc1, bfc1, Wfc2, bfc2):
    n, f_in = x.shape
    e = edge_index.shape[1]
    tr = 1000
    z32 = jnp.zeros((32,), jnp.float32)

    w1p = jnp.concatenate([Wc11, Wfc11, Wfc12, Wfc13,
                           jnp.zeros((f_in, 28), jnp.float32)], axis=1)
    bvec1 = jnp.concatenate([jnp.zeros((64,), jnp.float32), bfc11, bfc12,
                             bfc13, jnp.zeros((28,), jnp.float32)]
                            ).reshape(1, 128)
    w2p = jnp.concatenate([Wc21, Wfc21, Wfc22, Wfc23,
                           jnp.zeros((98, 28), jnp.float32)], axis=1)
    w2p = jnp.concatenate([w2p, jnp.zeros((30, 128), jnp.float32)], axis=0)
    bvec2 = jnp.concatenate([jnp.zeros((64,), jnp.float32), bfc21, bfc22,
                             bfc23, jnp.zeros((28,), jnp.float32)]
                            ).reshape(1, 128)
    bns = (bn1_g / jnp.sqrt(1.0 + 1e-5)).reshape(1, 98)
    bnb = bn1_b.reshape(1, 98)

    per = _NW * _CHUNK * _NBUF
    e_pad = ((e + per - 1) // per) * per
    npad = ((n + 127) // 128) * 128
    if npad == n:
        npad += 128
    src = edge_index[0]
    dst = edge_index[1]
    if e_pad != e:
        fill = jnp.full((e_pad - e,), n, jnp.int32)
        src = jnp.concatenate([src, fill])
        dst = jnp.concatenate([dst, fill])
    steps = e_pad // (_NW * _CHUNK)
    dummy = jnp.full((_NW, _NBUF, _CHUNK), n, jnp.int32)
    srcw = jnp.concatenate([src.reshape(_NW, steps, _CHUNK), dummy], axis=1)
    dstw = jnp.concatenate([dst.reshape(_NW, steps, _CHUNK), dummy], axis=1)
    zeros = jnp.zeros((npad, 64), jnp.float32)

    def pad_rows(a):
        if npad == n:
            return a
        return jnp.concatenate(
            [a, jnp.zeros((npad - n, 64), jnp.float32)], axis=0)

    xw1, d1 = _dense1(x, w1p, bvec1, tr)
    p = _sc_agg_impl(pad_rows(xw1), srcw, dstw, zeros, npad)
    xw2, d2 = _dense2(d1, p[:, :n], bc11.reshape(1, 64), bns, bnb,
                      w2p, bvec2, tr)
    q = _sc_agg_impl(pad_rows(xw2), srcw, dstw, zeros, npad)
    batch3 = batch.reshape(n // tr, tr, 1)
    return _pool(d2, q[:, :n], bc21.reshape(1, 64), batch3,
                 Wfc1, bfc1, Wfc2, bfc2, tr)


# speedup vs baseline: 1.1747x; 1.0143x over previous
import functools

import jax
import jax.numpy as jnp
from jax import lax
from jax.experimental import pallas as pl
from jax.experimental.pallas import tpu as pltpu
from jax.experimental.pallas import tpu_sc as plsc

_NC, _NS = 2, 16
_NW = _NC * _NS
_CHUNK = 80
_NBUF = 1
_NG = 64


def _dense1_body(x_ref, w_ref, bvec_ref, xw_ref, d_ref):
    y = jnp.dot(x_ref[...], w_ref[...], preferred_element_type=jnp.float32)
    yb = y + bvec_ref[...]
    xw_ref[...] = y[:, :64]
    a = jax.nn.relu(yb[:, 64:96])
    t = jnp.tanh(yb[:, 96:98]) * jnp.tanh(yb[:, 98:100])
    d_ref[...] = jnp.concatenate([a, t], axis=1)


def _dense1(x, w1p, bvec1, tr):
    n = x.shape[0]
    grid = n // tr
    return pl.pallas_call(
        _dense1_body,
        grid=(grid,),
        in_specs=[
            pl.BlockSpec((tr, 128), lambda i: (i, 0)),
            pl.BlockSpec((128, 128), lambda i: (0, 0)),
            pl.BlockSpec((1, 128), lambda i: (0, 0)),
        ],
        out_specs=[
            pl.BlockSpec((tr, 64), lambda i: (i, 0)),
            pl.BlockSpec((tr, 34), lambda i: (i, 0)),
        ],
        out_shape=[
            jax.ShapeDtypeStruct((n, 64), jnp.float32),
            jax.ShapeDtypeStruct((n, 34), jnp.float32),
        ],
        compiler_params=pltpu.CompilerParams(
            dimension_semantics=("parallel",)),
    )(x, w1p, bvec1)


def _dense2_body(d_ref, p_ref, bc_ref, bns_ref, bnb_ref, w_ref, bvec_ref,
                 xw_ref, d2_ref):
    spect = jax.nn.relu(p_ref[0] + p_ref[1] + bc_ref[...])
    tr = spect.shape[0]
    h1 = jnp.concatenate(
        [d_ref[:, :32], spect, d_ref[:, 32:34]], axis=1)
    h1 = h1 * bns_ref[...] + bnb_ref[...]
    h1p = jnp.concatenate([h1, jnp.zeros((tr, 30), jnp.float32)], axis=1)
    y = jnp.dot(h1p, w_ref[...], preferred_element_type=jnp.float32)
    yb = y + bvec_ref[...]
    xw_ref[...] = y[:, :64]
    a = jax.nn.relu(yb[:, 64:96])
    t = jnp.tanh(yb[:, 96:98]) * jnp.tanh(yb[:, 98:100])
    d2_ref[...] = jnp.concatenate([a, t], axis=1)


def _dense2(d1, p, bc11, bns, bnb, w2p, bvec2, tr):
    n = d1.shape[0]
    grid = n // tr
    return pl.pallas_call(
        _dense2_body,
        grid=(grid,),
        in_specs=[
            pl.BlockSpec((tr, 34), lambda i: (i, 0)),
            pl.BlockSpec((2, tr, 64), lambda i: (0, i, 0)),
            pl.BlockSpec((1, 64), lambda i: (0, 0)),
            pl.BlockSpec((1, 98), lambda i: (0, 0)),
            pl.BlockSpec((1, 98), lambda i: (0, 0)),
            pl.BlockSpec((128, 128), lambda i: (0, 0)),
            pl.BlockSpec((1, 128), lambda i: (0, 0)),
        ],
        out_specs=[
            pl.BlockSpec((tr, 64), lambda i: (i, 0)),
            pl.BlockSpec((tr, 34), lambda i: (i, 0)),
        ],
        out_shape=[
            jax.ShapeDtypeStruct((n, 64), jnp.float32),
            jax.ShapeDtypeStruct((n, 34), jnp.float32),
        ],
        compiler_params=pltpu.CompilerParams(
            dimension_semantics=("parallel",)),
    )(d1, p, bc11, bns, bnb, w2p, bvec2)


def _make_sc_agg(npad, steps, chunk):
    rows_per_tile = npad // _NS
    mesh = plsc.VectorSubcoreMesh(
        core_axis_name="c", subcore_axis_name="s",
        num_cores=_NC, num_subcores=_NS)

    @functools.partial(
        pl.kernel,
        out_type=jax.ShapeDtypeStruct((_NC, npad, 64), jnp.float32),
        mesh=mesh,
        scratch_types=[
            pltpu.VMEM((steps + _NBUF, chunk), jnp.int32),
            pltpu.VMEM((steps + _NBUF, chunk), jnp.int32),
            [pltpu.VMEM((chunk, 64), jnp.float32) for _ in range(_NBUF)],
            pltpu.VMEM_SHARED((npad, 64), jnp.float32),
            pltpu.VMEM_SHARED((npad, 64), jnp.float32),
            [pltpu.SemaphoreType.DMA for _ in range(_NBUF)],
        ],
        compiler_params=pltpu.CompilerParams(use_tc_tiling_on_sc=False),
    )
    def sc_agg(xw_hbm, src_hbm, dst_hbm, zeros_hbm, out_hbm,
               src_v, dst_v, rows, acc, tbl, gsem):
        c = lax.axis_index("c")
        s = lax.axis_index("s")
        wid = s * _NC + c
        r0 = s * rows_per_tile
        pltpu.sync_copy(zeros_hbm.at[pl.ds(r0, rows_per_tile)],
                        acc.at[pl.ds(r0, rows_per_tile)])
        pltpu.sync_copy(xw_hbm.at[pl.ds(r0, rows_per_tile)],
                        tbl.at[pl.ds(r0, rows_per_tile)])
        pltpu.sync_copy(src_hbm.at[wid], src_v)
        pltpu.sync_copy(dst_hbm.at[wid], dst_v)
        plsc.subcore_barrier()

        for b in range(_NBUF):
            pltpu.async_copy(tbl.at[src_v.at[b]], rows[b], gsem[b])

        def rnd(i, carry):
            j0 = i * _NBUF
            for b in range(_NBUF):
                pltpu.make_async_copy(
                    tbl.at[src_v.at[j0 + b]], rows[b], gsem[b]).wait()
                pltpu.sync_copy(rows[b], acc.at[dst_v.at[j0 + b]], add=True)
                pltpu.async_copy(tbl.at[src_v.at[j0 + _NBUF + b]],
                                 rows[b], gsem[b])
            return carry

        lax.fori_loop(0, steps // _NBUF, rnd, 0)
        for b in range(_NBUF):
            pltpu.make_async_copy(
                tbl.at[src_v.at[b]], rows[b], gsem[b]).wait()
        plsc.subcore_barrier()
        pltpu.sync_copy(acc.at[pl.ds(r0, rows_per_tile)],
                        out_hbm.at[c, pl.ds(r0, rows_per_tile)])

    return sc_agg


def _sc_agg_impl(xw, srcw, dstw, zeros, npad):
    steps = srcw.shape[1] - _NBUF
    return _make_sc_agg(npad, steps, srcw.shape[2])(xw, srcw, dstw, zeros)


def _pool_body(d_ref, q_ref, bc_ref, b_ref, w1_ref, b1_ref, w2_ref, b2_ref,
               o_ref, ps_ref, pm_ref):
    i = pl.program_id(0)

    @pl.when(i == 0)
    def _init():
        ps_ref[...] = jnp.zeros_like(ps_ref)
        pm_ref[...] = jnp.full_like(pm_ref, -jnp.inf)

    spect = jax.nn.relu(q_ref[0] + q_ref[1] + bc_ref[...])
    h2 = jnp.concatenate(
        [d_ref[:, :32], spect, d_ref[:, 32:34]], axis=1)
    b2 = b_ref[0]
    tr = h2.shape[0]
    onehot = (b2 == lax.broadcasted_iota(jnp.int32, (tr, _NG), 1)
              ).astype(jnp.float32)
    ps_ref[...] += lax.dot_general(
        onehot, h2, (((0,), (0,)), ((), ())),
        preferred_element_type=jnp.float32)
    m = h2
    dshift = 1
    while dshift < tr:
        mprev = jnp.concatenate(
            [jnp.full((dshift, 98), -jnp.inf, jnp.float32), m[:-dshift]],
            axis=0)
        bprev = jnp.concatenate(
            [jnp.full((dshift, 1), -1, jnp.int32), b2[:-dshift]], axis=0)
        m = jnp.where(bprev == b2, jnp.maximum(m, mprev), m)
        dshift *= 2
    bnext = jnp.concatenate(
        [b2[1:], jnp.full((1, 1), -1, jnp.int32)], axis=0)
    is_end = (b2 != bnext).astype(jnp.float32)
    sel = onehot * is_end
    contrib = lax.dot_general(
        sel, m, (((0,), (0,)), ((), ())),
        preferred_element_type=jnp.float32)
    present = jnp.sum(sel, axis=0)[:, None] > 0.0
    pm_ref[...] = jnp.maximum(
        pm_ref[...], jnp.where(present, contrib, -jnp.inf))

    @pl.when(i == pl.num_programs(0) - 1)
    def _head():
        pooled = jnp.concatenate([ps_ref[...], pm_ref[...]], axis=1)
        z1 = jax.nn.relu(
            jnp.dot(pooled, w1_ref[...], preferred_element_type=jnp.float32)
            + b1_ref[...])
        z = (jnp.dot(z1, w2_ref[...], preferred_element_type=jnp.float32)
             + b2_ref[...])
        m = jnp.max(z, axis=1, keepdims=True)
        lse = jnp.log(jnp.sum(jnp.exp(z - m), axis=1, keepdims=True)) + m
        o_ref[...] = z - lse


def _pool(d2, q, bc21, batch3, wfc1, bfc1, wfc2, bfc2, tr):
    n = d2.shape[0]
    grid = n // tr
    return pl.pallas_call(
        _pool_body,
        grid=(grid,),
        in_specs=[
            pl.BlockSpec((tr, 34), lambda i: (i, 0)),
            pl.BlockSpec((2, tr, 64), lambda i: (0, i, 0)),
            pl.BlockSpec((1, 64), lambda i: (0, 0)),
            pl.BlockSpec((1, tr, 1), lambda i: (i, 0, 0)),
            pl.BlockSpec((196, 100), lambda i: (0, 0)),
            pl.BlockSpec((1, 100), lambda i: (0, 0)),
            pl.BlockSpec((100, 2), lambda i: (0, 0)),
            pl.BlockSpec((1, 2), lambda i: (0, 0)),
        ],
        out_specs=pl.BlockSpec((_NG, 2), lambda i: (0, 0)),
        out_shape=jax.ShapeDtypeStruct((_NG, 2), jnp.float32),
        scratch_shapes=[
            pltpu.VMEM((_NG, 98), jnp.float32),
            pltpu.VMEM((_NG, 98), jnp.float32),
        ],
        compiler_params=pltpu.CompilerParams(
            dimension_semantics=("arbitrary",)),
    )(d2, q, bc21, batch3, wfc1, bfc1.reshape(1, -1), wfc2,
      bfc2.reshape(1, -1))


def kernel(x, edge_index, batch, Wc11, bc11, Wfc11, bfc11, Wfc12, bfc12,
           Wfc13, bfc13, bn1_g, bn1_b, Wc21, bc21, Wfc21, bfc21, Wfc22,
           bfc22, Wfc23, bfc23, Wfc1, bfc1, Wfc2, bfc2):
    n, f_in = x.shape
    e = edge_index.shape[1]
    tr = 2000
    z32 = jnp.zeros((32,), jnp.float32)

    w1p = jnp.concatenate([Wc11, Wfc11, Wfc12, Wfc13,
                           jnp.zeros((f_in, 28), jnp.float32)], axis=1)
    bvec1 = jnp.concatenate([jnp.zeros((64,), jnp.float32), bfc11, bfc12,
                             bfc13, jnp.zeros((28,), jnp.float32)]
                            ).reshape(1, 128)
    w2p = jnp.concatenate([Wc21, Wfc21, Wfc22, Wfc23,
                           jnp.zeros((98, 28), jnp.float32)], axis=1)
    w2p = jnp.concatenate([w2p, jnp.zeros((30, 128), jnp.float32)], axis=0)
    bvec2 = jnp.concatenate([jnp.zeros((64,), jnp.float32), bfc21, bfc22,
                             bfc23, jnp.zeros((28,), jnp.float32)]
                            ).reshape(1, 128)
    bns = (bn1_g / jnp.sqrt(1.0 + 1e-5)).reshape(1, 98)
    bnb = bn1_b.reshape(1, 98)

    per = _NW * _CHUNK * _NBUF
    e_pad = ((e + per - 1) // per) * per
    npad = ((n + 127) // 128) * 128
    if npad == n:
        npad += 128
    src = edge_index[0]
    dst = edge_index[1]
    if e_pad != e:
        fill = jnp.full((e_pad - e,), n, jnp.int32)
        src = jnp.concatenate([src, fill])
        dst = jnp.concatenate([dst, fill])
    steps = e_pad // (_NW * _CHUNK)
    dummy = jnp.full((_NW, _NBUF, _CHUNK), n, jnp.int32)
    srcw = jnp.concatenate([src.reshape(_NW, steps, _CHUNK), dummy], axis=1)
    dstw = jnp.concatenate([dst.reshape(_NW, steps, _CHUNK), dummy], axis=1)
    zeros = jnp.zeros((npad, 64), jnp.float32)

    def pad_rows(a):
        if npad == n:
            return a
        return jnp.concatenate(
            [a, jnp.zeros((npad - n, 64), jnp.float32)], axis=0)

    xw1, d1 = _dense1(x, w1p, bvec1, tr)
    p = _sc_agg_impl(pad_rows(xw1), srcw, dstw, zeros, npad)
    xw2, d2 = _dense2(d1, p[:, :n], bc11.reshape(1, 64), bns, bnb,
                      w2p, bvec2, tr)
    q = _sc_agg_impl(pad_rows(xw2), srcw, dstw, zeros, npad)
    batch3 = batch.reshape(n // tr, tr, 1)
    return _pool(d2, q[:, :n], bc21.reshape(1, 64), batch3,
                 Wfc1, bfc1, Wfc2, bfc2, tr)
